# baseline (speedup 1.0000x reference)
import functools

import jax
import jax.numpy as jnp
from jax import lax
from jax.experimental import pallas as pl
from jax.experimental.pallas import tpu as pltpu
from jax.experimental.pallas import tpu_sc as plsc

N = 10000
E = 320000
DIN = 128
HID = 8
HEADS = 8
OUT = 16
G = 64

C = 64
NCHUNK = E // C
NW = 32
KMAX = (NCHUNK + NW - 1) // NW
RPT = 624
RPT_LAST = N - 15 * RPT


def _tile_rows_copy(sid, src_at, dst_at):
    @pl.when(sid < 15)
    def _():
        pltpu.sync_copy(src_at(sid * RPT, RPT), dst_at(sid * RPT, RPT))

    @pl.when(sid == 15)
    def _():
        pltpu.sync_copy(src_at(15 * RPT, RPT_LAST), dst_at(15 * RPT, RPT_LAST))

_f32 = jnp.float32


def _sc_body_wide(src_h, dst_h, hs_h, ed_h, zeros_h, out_h,
                  is0, is1, id0, id1, ic0, ic1, rows0, rows1, edr0, edr1,
                  outr0, outr1, si0, si1, sg0, sg1, ss0, ss1, acc_sh):
    cid = lax.axis_index("c")
    sid = lax.axis_index("s")
    wid = sid * 2 + cid

    isv = (is0, is1)
    idv = (id0, id1)
    idsc = (ic0, ic1)
    rows = (rows0, rows1)
    edr = (edr0, edr1)
    outr = (outr0, outr1)
    si = (si0, si1)
    ss = (ss0, ss1)

    _tile_rows_copy(sid,
                    lambda o, n: zeros_h.at[pl.ds(o, n)],
                    lambda o, n: acc_sh.at[pl.ds(o, n)])
    plsc.subcore_barrier()

    def cbase(k):
        return jnp.minimum(wid + NW * k, NCHUNK - 1) * C

    def fire_idx(k, b):
        base = cbase(k)
        pltpu.async_copy(src_h.at[pl.ds(base, C)], isv[b], si[b])
        pltpu.async_copy(dst_h.at[pl.ds(base, C)], idv[b], si[b])

    def wait_idx(b):
        pltpu.make_async_copy(src_h.at[pl.ds(0, C)], isv[b], si[b]).wait()
        pltpu.make_async_copy(dst_h.at[pl.ds(0, C)], idv[b], si[b]).wait()

    def wait_scat(b):
        pltpu.make_async_copy(outr[b], acc_sh.at[idsc[b]], ss[b]).wait()

    fire_idx(0, 0)
    fire_idx(1, 1)
    wait_idx(0)
    g1 = pltpu.async_copy(hs_h.at[is0], rows0, sg0)
    g2 = pltpu.async_copy(ed_h.at[id0], edr0, sg0)
    g1.wait()
    g2.wait()

    def half(k, b):
        bn = 1 - b

        @pl.when(k < KMAX)
        def _():
            wait_idx(bn)
            cg1 = pltpu.async_copy(hs_h.at[isv[bn]], rows[bn], sg1 if bn else sg0)
            cg2 = pltpu.async_copy(ed_h.at[idv[bn]], edr[bn], sg1 if bn else sg0)

            @pl.when(k >= 2)
            def _():
                wait_scat(b)

            def edge(e, c2):
                for blk in range(4):
                    ev = (rows[b][e, pl.ds(64 + 16 * blk, 16)]
                          + edr[b][e, pl.ds(16 * blk, 16)])
                    p = jnp.exp(jnp.maximum(ev, 0.2 * ev))
                    outr[b][e, pl.ds(16 * blk, 16)] = (
                        rows[b][e, pl.ds(16 * blk, 16)] * p)
                    outr[b][e, pl.ds(64 + 16 * blk, 16)] = p
                return c2

            lax.fori_loop(0, C, edge, 0)

            for q in range(C // 16):
                idsc[b][pl.ds(16 * q, 16)] = idv[b][pl.ds(16 * q, 16)]

            @pl.when(wid + NW * k < NCHUNK)
            def _():
                pltpu.async_copy(outr[b], acc_sh.at[idsc[b]], ss[b], add=True)

            @pl.when(k < KMAX - 1)
            def _():
                fire_idx(k + 2, b)

            cg1.wait()
            cg2.wait()

    def pair(k2, carry):
        half(2 * k2, 0)
        half(2 * k2 + 1, 1)
        return carry

    lax.fori_loop(0, (KMAX + 1) // 2, pair, 0)
    wait_scat((KMAX - 2) % 2)

    @pl.when(wid + NW * (KMAX - 1) < NCHUNK)
    def _():
        wait_scat((KMAX - 1) % 2)

    plsc.subcore_barrier()
    _tile_rows_copy(sid,
                    lambda o, n: acc_sh.at[pl.ds(o, n)],
                    lambda o, n: out_h.at[pl.ds(cid * N + o, n)])


def _make_sc_edge(width_hs, width_ed, body):
    mesh = plsc.VectorSubcoreMesh(core_axis_name="c", subcore_axis_name="s")
    return pl.kernel(
        body,
        out_type=jax.ShapeDtypeStruct((2 * N, width_hs), _f32),
        mesh=mesh,
        compiler_params=pltpu.CompilerParams(use_tc_tiling_on_sc=False),
        scratch_types=(
            [pltpu.VMEM((C,), jnp.int32)] * 6
            + [pltpu.VMEM((C, width_hs), _f32)] * 2
            + [pltpu.VMEM((C, width_ed), _f32)] * 2
            + [pltpu.VMEM((C, width_hs), _f32)] * 2
            + [pltpu.SemaphoreType.DMA] * 6
            + [pltpu.VMEM_SHARED((N, width_hs), _f32)]
        ),
    )


@functools.lru_cache(maxsize=None)
def _sc_kernel():
    return _make_sc_edge(128, 64, _sc_body_wide)


def _t1_body(x_ref, w_ref, hs_ref, ed_ref):
    y = jnp.dot(x_ref[...], w_ref[...], preferred_element_type=_f32)
    hs_ref[...] = y[:, :128]
    ed_ref[...] = y[:, 128:192]


def _t2_body(a0_ref, a1_ref, b_ref, w_ref, hs_ref, ed_ref):
    s = a0_ref[...] + a1_ref[...]
    g = s[:, :64] / (s[:, 64:128] + 1e-16) + b_ref[...]
    g = jnp.where(g > 0, g, jnp.exp(jnp.minimum(g, 0.0)) - 1.0)
    y = jnp.dot(g, w_ref[...], preferred_element_type=_f32)
    hs_ref[...] = y[:, :128]
    ed_ref[...] = y[:, 128:192]


def _t4_body(a0_ref, a1_ref, batch_ref, b3_ref, wc_ref, bc_ref, out_ref):
    s = a0_ref[...] + a1_ref[...]
    h = s[:, :8] / (s[:, 64:72] + 1e-16) + b3_ref[...]
    gid = lax.broadcasted_iota(jnp.int32, (N, G), 1)
    oh = (batch_ref[...] == gid).astype(_f32)
    sums = lax.dot_general(oh, h, (((0,), (0,)), ((), ())),
                           preferred_element_type=_f32)
    ones = jnp.ones((N, 1), _f32)
    cnt = lax.dot_general(oh, ones, (((0,), (0,)), ((), ())),
                          preferred_element_type=_f32)
    pooled = sums / jnp.maximum(cnt, 1.0)
    logits = jnp.dot(pooled, wc_ref[...], preferred_element_type=_f32) + bc_ref[...]
    mx = jnp.max(logits, axis=1, keepdims=True)
    lse = mx + jnp.log(jnp.sum(jnp.exp(logits - mx), axis=1, keepdims=True))
    out_ref[...] = logits - lse


_RB = 1000
_NB = N // _RB


def _full(shape):
    return pl.BlockSpec(shape, lambda i: tuple(0 for _ in shape))


def _rows(width):
    return pl.BlockSpec((_RB, width), lambda i: (i, 0))


_pair_out = dict(
    out_specs=[_rows(128), _rows(64)],
    out_shape=[jax.ShapeDtypeStruct((N, 128), _f32),
               jax.ShapeDtypeStruct((N, 64), _f32)],
)

_t1 = pl.pallas_call(
    _t1_body,
    grid=(_NB,),
    in_specs=[_rows(128), _full((128, 192))],
    **_pair_out,
)

_t2 = pl.pallas_call(
    _t2_body,
    grid=(_NB,),
    in_specs=[_rows(128), _rows(128), _full((1, 64)), _full((64, 192))],
    **_pair_out,
)

_t4 = pl.pallas_call(
    _t4_body,
    grid=(1,),
    in_specs=[_full((N, 128)), _full((N, 128)), _full((N, 1)),
              _full((1, 8)), _full((8, 16)), _full((1, 16))],
    out_specs=_full((G, OUT)),
    out_shape=jax.ShapeDtypeStruct((G, OUT), _f32),
)


def _compact_mat(a):
    hh = a.shape[0] * a.shape[1]
    hid = a.shape[1]
    ii = jnp.arange(hh)
    jj = jnp.arange(a.shape[0])
    mask = (ii[:, None] // hid) == jj[None, :]
    return mask.astype(_f32) * a.reshape(hh)[:, None]


def _expand_mat(a):
    hh = a.shape[0] * a.shape[1]
    hid = a.shape[1]
    ii = jnp.arange(hh)
    mask = (ii[:, None] // hid) == (ii[None, :] // hid)
    return mask.astype(_f32) * a.reshape(hh)[:, None]


def kernel(x, edge_index, batch, W1, a_s1, a_d1, b1, W2, a_s2, a_d2, b2,
           W3, a_s3, a_d3, b3, Wc, bc):
    src = edge_index[0]
    dst = edge_index[1]

    w1c = jnp.concatenate(
        [W1, W1 @ _expand_mat(a_s1), W1 @ _expand_mat(a_d1)], axis=1)
    w2c = jnp.concatenate(
        [W2, W2 @ _expand_mat(a_s2), W2 @ _expand_mat(a_d2)], axis=1)
    es3 = a_s3.reshape(HID, 1) @ jnp.ones((1, 64), _f32)
    ed3 = a_d3.reshape(HID, 1) @ jnp.ones((1, 64), _f32)
    w3c = jnp.concatenate(
        [jnp.tile(W3, (1, HEADS)), W3 @ es3, W3 @ ed3], axis=1)

    zw = jnp.zeros((N, 128), _f32)

    sc_edge = _sc_kernel()

    hs1, ed1 = _t1(x, w1c)
    acc1 = sc_edge(src, dst, hs1, ed1, zw)
    hs2, ed2 = _t2(acc1[:N], acc1[N:], b1.reshape(1, 64), w2c)
    acc2 = sc_edge(src, dst, hs2, ed2, zw)
    hs3, ed3t = _t2(acc2[:N], acc2[N:], b2.reshape(1, 64), w3c)
    acc3 = sc_edge(src, dst, hs3, ed3t, zw)
    out = _t4(acc3[:N], acc3[N:], batch.reshape(N, 1), b3.reshape(1, 8),
              Wc, bc.reshape(1, OUT))
    return out


# speedup vs baseline: 2.6285x; 2.6285x over previous
import functools

import jax
import jax.numpy as jnp
from jax import lax
from jax.experimental import pallas as pl
from jax.experimental.pallas import tpu as pltpu
from jax.experimental.pallas import tpu_sc as plsc

N = 10000
E = 320000
DIN = 128
HID = 8
HEADS = 8
OUT = 16
G = 64

C = 64
NCHUNK = E // C
NW = 32
KMAX = (NCHUNK + NW - 1) // NW
RPT = 624
RPT_LAST = N - 15 * RPT


def _tile_rows_copy(sid, src_at, dst_at):
    @pl.when(sid < 15)
    def _():
        pltpu.sync_copy(src_at(sid * RPT, RPT), dst_at(sid * RPT, RPT))

    @pl.when(sid == 15)
    def _():
        pltpu.sync_copy(src_at(15 * RPT, RPT_LAST), dst_at(15 * RPT, RPT_LAST))

_f32 = jnp.float32


def _sc_body_wide(src_h, dst_h, hs_h, ed_h, zeros_h, out_h,
                  is0, is1, id0, id1, ic0, ic1, rows0, rows1, edr0, edr1,
                  outr0, outr1, si0, si1, sg0, sg1, ss0, ss1, acc_sh):
    cid = lax.axis_index("c")
    sid = lax.axis_index("s")
    wid = sid * 2 + cid

    isv = (is0, is1)
    idv = (id0, id1)
    idsc = (ic0, ic1)
    rows = (rows0, rows1)
    edr = (edr0, edr1)
    outr = (outr0, outr1)
    si = (si0, si1)
    ss = (ss0, ss1)

    _tile_rows_copy(sid,
                    lambda o, n: zeros_h.at[pl.ds(o, n)],
                    lambda o, n: acc_sh.at[pl.ds(o, n)])
    plsc.subcore_barrier()

    def cbase(k):
        return jnp.minimum(wid + NW * k, NCHUNK - 1) * C

    def fire_idx(k, b):
        base = cbase(k)
        pltpu.async_copy(src_h.at[pl.ds(base, C)], isv[b], si[b])
        pltpu.async_copy(dst_h.at[pl.ds(base, C)], idv[b], si[b])

    def wait_idx(b):
        pltpu.make_async_copy(src_h.at[pl.ds(0, C)], isv[b], si[b]).wait()
        pltpu.make_async_copy(dst_h.at[pl.ds(0, C)], idv[b], si[b]).wait()

    def wait_scat(b):
        pltpu.make_async_copy(outr[b], acc_sh.at[idsc[b]], ss[b]).wait()

    fire_idx(0, 0)
    fire_idx(1, 1)
    wait_idx(0)
    g1 = pltpu.async_copy(hs_h.at[is0], rows0, sg0)
    g2 = pltpu.async_copy(ed_h.at[id0], edr0, sg0)
    g1.wait()
    g2.wait()

    def half(k, b):
        bn = 1 - b

        @pl.when(k < KMAX)
        def _():
            wait_idx(bn)
            cg1 = pltpu.async_copy(hs_h.at[isv[bn]], rows[bn], sg1 if bn else sg0)
            cg2 = pltpu.async_copy(ed_h.at[idv[bn]], edr[bn], sg1 if bn else sg0)

            @pl.when(k >= 2)
            def _():
                wait_scat(b)

            def edge(e, c2):
                for blk in range(4):
                    ev = (rows[b][e, pl.ds(64 + 16 * blk, 16)]
                          + edr[b][e, pl.ds(16 * blk, 16)])
                    p = jnp.exp(jnp.maximum(ev, 0.2 * ev))
                    outr[b][e, pl.ds(16 * blk, 16)] = (
                        rows[b][e, pl.ds(16 * blk, 16)] * p)
                    outr[b][e, pl.ds(64 + 16 * blk, 16)] = p
                return c2

            lax.fori_loop(0, C, edge, 0)

            for q in range(C // 16):
                idsc[b][pl.ds(16 * q, 16)] = idv[b][pl.ds(16 * q, 16)]

            @pl.when(wid + NW * k < NCHUNK)
            def _():
                pltpu.async_copy(outr[b], acc_sh.at[idsc[b]], ss[b], add=True)

            @pl.when(k < KMAX - 1)
            def _():
                fire_idx(k + 2, b)

            cg1.wait()
            cg2.wait()

    def pair(k2, carry):
        half(2 * k2, 0)
        half(2 * k2 + 1, 1)
        return carry

    lax.fori_loop(0, (KMAX + 1) // 2, pair, 0)
    wait_scat((KMAX - 2) % 2)

    @pl.when(wid + NW * (KMAX - 1) < NCHUNK)
    def _():
        wait_scat((KMAX - 1) % 2)

    plsc.subcore_barrier()
    _tile_rows_copy(sid,
                    lambda o, n: acc_sh.at[pl.ds(o, n)],
                    lambda o, n: out_h.at[pl.ds(cid * N + o, n)])


def _make_sc_edge(width_hs, width_ed, body):
    mesh = plsc.VectorSubcoreMesh(core_axis_name="c", subcore_axis_name="s")
    return pl.kernel(
        body,
        out_type=jax.ShapeDtypeStruct((2 * N, width_hs), _f32),
        mesh=mesh,
        scratch_types=(
            [pltpu.VMEM((C,), jnp.int32)] * 6
            + [pltpu.VMEM((C, width_hs), _f32)] * 2
            + [pltpu.VMEM((C, width_ed), _f32)] * 2
            + [pltpu.VMEM((C, width_hs), _f32)] * 2
            + [pltpu.SemaphoreType.DMA] * 6
            + [pltpu.VMEM_SHARED((N, width_hs), _f32)]
        ),
    )


@functools.lru_cache(maxsize=None)
def _sc_kernel():
    return _make_sc_edge(128, 128, _sc_body_wide)


def _t1_body(x_ref, w_ref, hs_ref, ed_ref):
    y = jnp.dot(x_ref[...], w_ref[...], preferred_element_type=_f32)
    hs_ref[...] = y[:, :128]
    ed_ref[...] = y[:, 128:256]


def _t2_body(a0_ref, a1_ref, b_ref, w_ref, hs_ref, ed_ref):
    s = a0_ref[...] + a1_ref[...]
    g = s[:, :64] / (s[:, 64:128] + 1e-16) + b_ref[...]
    g = jnp.where(g > 0, g, jnp.exp(jnp.minimum(g, 0.0)) - 1.0)
    y = jnp.dot(g, w_ref[...], preferred_element_type=_f32)
    hs_ref[...] = y[:, :128]
    ed_ref[...] = y[:, 128:256]


def _t4_body(a0_ref, a1_ref, batch_ref, b3_ref, wc_ref, bc_ref, out_ref):
    s = a0_ref[...] + a1_ref[...]
    h = s[:, :8] / (s[:, 64:72] + 1e-16) + b3_ref[...]
    gid = lax.broadcasted_iota(jnp.int32, (N, G), 1)
    oh = (batch_ref[...] == gid).astype(_f32)
    sums = lax.dot_general(oh, h, (((0,), (0,)), ((), ())),
                           preferred_element_type=_f32)
    ones = jnp.ones((N, 1), _f32)
    cnt = lax.dot_general(oh, ones, (((0,), (0,)), ((), ())),
                          preferred_element_type=_f32)
    pooled = sums / jnp.maximum(cnt, 1.0)
    logits = jnp.dot(pooled, wc_ref[...], preferred_element_type=_f32) + bc_ref[...]
    mx = jnp.max(logits, axis=1, keepdims=True)
    lse = mx + jnp.log(jnp.sum(jnp.exp(logits - mx), axis=1, keepdims=True))
    out_ref[...] = logits - lse


_RB = 1000
_NB = N // _RB


def _full(shape):
    return pl.BlockSpec(shape, lambda i: tuple(0 for _ in shape))


def _rows(width):
    return pl.BlockSpec((_RB, width), lambda i: (i, 0))


_pair_out = dict(
    out_specs=[_rows(128), _rows(128)],
    out_shape=[jax.ShapeDtypeStruct((N, 128), _f32),
               jax.ShapeDtypeStruct((N, 128), _f32)],
)

_t1 = pl.pallas_call(
    _t1_body,
    grid=(_NB,),
    in_specs=[_rows(128), _full((128, 256))],
    **_pair_out,
)

_rows_hi = pl.BlockSpec((_RB, 128), lambda i: (i + _NB, 0))

_t2 = pl.pallas_call(
    _t2_body,
    grid=(_NB,),
    in_specs=[_rows(128), _rows_hi, _full((1, 64)), _full((64, 256))],
    **_pair_out,
)

_t4 = pl.pallas_call(
    _t4_body,
    grid=(1,),
    in_specs=[pl.BlockSpec((N, 128), lambda i: (0, 0)),
              pl.BlockSpec((N, 128), lambda i: (1, 0)),
              _full((N, 1)),
              _full((1, 8)), _full((8, 16)), _full((1, 16))],
    out_specs=_full((G, OUT)),
    out_shape=jax.ShapeDtypeStruct((G, OUT), _f32),
)


def _compact_mat(a):
    hh = a.shape[0] * a.shape[1]
    hid = a.shape[1]
    ii = jnp.arange(hh)
    jj = jnp.arange(a.shape[0])
    mask = (ii[:, None] // hid) == jj[None, :]
    return mask.astype(_f32) * a.reshape(hh)[:, None]


def _expand_mat(a):
    hh = a.shape[0] * a.shape[1]
    hid = a.shape[1]
    ii = jnp.arange(hh)
    mask = (ii[:, None] // hid) == (ii[None, :] // hid)
    return mask.astype(_f32) * a.reshape(hh)[:, None]


def kernel(x, edge_index, batch, W1, a_s1, a_d1, b1, W2, a_s2, a_d2, b2,
           W3, a_s3, a_d3, b3, Wc, bc):
    src = edge_index[0]
    dst = edge_index[1]

    z64 = jnp.zeros((64, 64), _f32)
    w1c = jnp.concatenate(
        [W1, W1 @ _expand_mat(a_s1), W1 @ _expand_mat(a_d1),
         jnp.zeros((DIN, 64), _f32)], axis=1)
    w2c = jnp.concatenate(
        [W2, W2 @ _expand_mat(a_s2), W2 @ _expand_mat(a_d2), z64], axis=1)
    es3 = a_s3.reshape(HID, 1) @ jnp.ones((1, 64), _f32)
    ed3 = a_d3.reshape(HID, 1) @ jnp.ones((1, 64), _f32)
    w3c = jnp.concatenate(
        [jnp.tile(W3, (1, HEADS)), W3 @ es3, W3 @ ed3, z64], axis=1)

    zw = jnp.zeros((N, 128), _f32)

    sc_edge = _sc_kernel()

    hs1, ed1 = _t1(x, w1c)
    acc1 = sc_edge(src, dst, hs1, ed1, zw)
    hs2, ed2 = _t2(acc1, acc1, b1.reshape(1, 64), w2c)
    acc2 = sc_edge(src, dst, hs2, ed2, zw)
    hs3, ed3t = _t2(acc2, acc2, b2.reshape(1, 64), w3c)
    acc3 = sc_edge(src, dst, hs3, ed3t, zw)
    out = _t4(acc3, acc3, batch.reshape(N, 1), b3.reshape(1, 8),
              Wc, bc.reshape(1, OUT))
    return out


# speedup vs baseline: 2.6612x; 1.0124x over previous
import functools

import jax
import jax.numpy as jnp
from jax import lax
from jax.experimental import pallas as pl
from jax.experimental.pallas import tpu as pltpu
from jax.experimental.pallas import tpu_sc as plsc

N = 10000
E = 320000
DIN = 128
HID = 8
HEADS = 8
OUT = 16
G = 64

C = 64
NCHUNK = E // C
NW = 32
KMAX = (NCHUNK + NW - 1) // NW
RPT = 624
RPT_LAST = N - 15 * RPT


def _tile_rows_copy(sid, src_at, dst_at):
    @pl.when(sid < 15)
    def _():
        pltpu.sync_copy(src_at(sid * RPT, RPT), dst_at(sid * RPT, RPT))

    @pl.when(sid == 15)
    def _():
        pltpu.sync_copy(src_at(15 * RPT, RPT_LAST), dst_at(15 * RPT, RPT_LAST))

_f32 = jnp.float32


def _sc_body_wide(src_h, dst_h, hs_h, ed_h, out_h,
                  is0, is1, id0, id1, ic0, ic1, rows0, rows1, edr0, edr1,
                  outr0, outr1, si0, si1, sg0, sg1, ss0, ss1, acc_sh):
    cid = lax.axis_index("c")
    sid = lax.axis_index("s")
    wid = sid * 2 + cid

    isv = (is0, is1)
    idv = (id0, id1)
    idsc = (ic0, ic1)
    rows = (rows0, rows1)
    edr = (edr0, edr1)
    outr = (outr0, outr1)
    si = (si0, si1)
    ss = (ss0, ss1)

    def zrow(r, c):
        for q in range(8):
            outr0[r, pl.ds(16 * q, 16)] = jnp.zeros((16,), _f32)
        return c

    lax.fori_loop(0, C, zrow, 0)

    @pl.when(sid < 15)
    def _():
        for j in range(RPT // C):
            pltpu.sync_copy(outr0, acc_sh.at[pl.ds(sid * RPT + C * j, C)])
        rem = RPT % C
        pltpu.sync_copy(outr0.at[pl.ds(0, rem)],
                        acc_sh.at[pl.ds(sid * RPT + RPT - rem, rem)])

    @pl.when(sid == 15)
    def _():
        for j in range(RPT_LAST // C):
            pltpu.sync_copy(outr0, acc_sh.at[pl.ds(15 * RPT + C * j, C)])

    plsc.subcore_barrier()

    def cbase(k):
        return jnp.minimum(wid + NW * k, NCHUNK - 1) * C

    def fire_idx(k, b):
        base = cbase(k)
        pltpu.async_copy(src_h.at[pl.ds(base, C)], isv[b], si[b])
        pltpu.async_copy(dst_h.at[pl.ds(base, C)], idv[b], si[b])

    def wait_idx(b):
        pltpu.make_async_copy(src_h.at[pl.ds(0, C)], isv[b], si[b]).wait()
        pltpu.make_async_copy(dst_h.at[pl.ds(0, C)], idv[b], si[b]).wait()

    def wait_scat(b):
        pltpu.make_async_copy(outr[b], acc_sh.at[idsc[b]], ss[b]).wait()

    fire_idx(0, 0)
    fire_idx(1, 1)
    wait_idx(0)
    g1 = pltpu.async_copy(hs_h.at[is0], rows0, sg0)
    g2 = pltpu.async_copy(ed_h.at[id0], edr0, sg0)
    g1.wait()
    g2.wait()

    def half(k, b):
        bn = 1 - b

        @pl.when(k < KMAX)
        def _():
            wait_idx(bn)
            cg1 = pltpu.async_copy(hs_h.at[isv[bn]], rows[bn], sg1 if bn else sg0)
            cg2 = pltpu.async_copy(ed_h.at[idv[bn]], edr[bn], sg1 if bn else sg0)

            @pl.when(k >= 2)
            def _():
                wait_scat(b)

            def edge(e, c2):
                for blk in range(4):
                    ev = (rows[b][e, pl.ds(64 + 16 * blk, 16)]
                          + edr[b][e, pl.ds(16 * blk, 16)])
                    p = jnp.exp(jnp.maximum(ev, 0.2 * ev))
                    outr[b][e, pl.ds(16 * blk, 16)] = (
                        rows[b][e, pl.ds(16 * blk, 16)] * p)
                    outr[b][e, pl.ds(64 + 16 * blk, 16)] = p
                return c2

            lax.fori_loop(0, C, edge, 0)

            for q in range(C // 16):
                idsc[b][pl.ds(16 * q, 16)] = idv[b][pl.ds(16 * q, 16)]

            @pl.when(wid + NW * k < NCHUNK)
            def _():
                pltpu.async_copy(outr[b], acc_sh.at[idsc[b]], ss[b], add=True)

            @pl.when(k < KMAX - 1)
            def _():
                fire_idx(k + 2, b)

            cg1.wait()
            cg2.wait()

    def pair(k2, carry):
        half(2 * k2, 0)
        half(2 * k2 + 1, 1)
        return carry

    lax.fori_loop(0, (KMAX + 1) // 2, pair, 0)
    wait_scat((KMAX - 2) % 2)

    @pl.when(wid + NW * (KMAX - 1) < NCHUNK)
    def _():
        wait_scat((KMAX - 1) % 2)

    plsc.subcore_barrier()
    _tile_rows_copy(sid,
                    lambda o, n: acc_sh.at[pl.ds(o, n)],
                    lambda o, n: out_h.at[pl.ds(cid * N + o, n)])


def _make_sc_edge(width_hs, width_ed, body):
    mesh = plsc.VectorSubcoreMesh(core_axis_name="c", subcore_axis_name="s")
    return pl.kernel(
        body,
        out_type=jax.ShapeDtypeStruct((2 * N, width_hs), _f32),
        mesh=mesh,
        scratch_types=(
            [pltpu.VMEM((C,), jnp.int32)] * 6
            + [pltpu.VMEM((C, width_hs), _f32)] * 2
            + [pltpu.VMEM((C, width_ed), _f32)] * 2
            + [pltpu.VMEM((C, width_hs), _f32)] * 2
            + [pltpu.SemaphoreType.DMA] * 6
            + [pltpu.VMEM_SHARED((N, width_hs), _f32)]
        ),
    )


@functools.lru_cache(maxsize=None)
def _sc_kernel():
    return _make_sc_edge(128, 128, _sc_body_wide)


def _t1_body(x_ref, w_ref, hs_ref, ed_ref):
    y = jnp.dot(x_ref[...], w_ref[...], preferred_element_type=_f32)
    hs_ref[...] = y[:, :128]
    ed_ref[...] = y[:, 128:256]


def _t2_body(a0_ref, a1_ref, b_ref, w_ref, hs_ref, ed_ref):
    s = a0_ref[...] + a1_ref[...]
    g = s[:, :64] / (s[:, 64:128] + 1e-16) + b_ref[...]
    g = jnp.where(g > 0, g, jnp.exp(jnp.minimum(g, 0.0)) - 1.0)
    y = jnp.dot(g, w_ref[...], preferred_element_type=_f32)
    hs_ref[...] = y[:, :128]
    ed_ref[...] = y[:, 128:256]


def _t4_body(a0_ref, a1_ref, batch_ref, b3_ref, wc_ref, bc_ref, out_ref):
    s = a0_ref[...] + a1_ref[...]
    h = s[:, :8] / (s[:, 64:72] + 1e-16) + b3_ref[...]
    gid = lax.broadcasted_iota(jnp.int32, (N, G), 1)
    oh = (batch_ref[...] == gid).astype(_f32)
    sums = lax.dot_general(oh, h, (((0,), (0,)), ((), ())),
                           preferred_element_type=_f32)
    ones = jnp.ones((N, 1), _f32)
    cnt = lax.dot_general(oh, ones, (((0,), (0,)), ((), ())),
                          preferred_element_type=_f32)
    pooled = sums / jnp.maximum(cnt, 1.0)
    logits = jnp.dot(pooled, wc_ref[...], preferred_element_type=_f32) + bc_ref[...]
    mx = jnp.max(logits, axis=1, keepdims=True)
    lse = mx + jnp.log(jnp.sum(jnp.exp(logits - mx), axis=1, keepdims=True))
    out_ref[...] = logits - lse


_RB = 1000
_NB = N // _RB


def _full(shape):
    return pl.BlockSpec(shape, lambda i: tuple(0 for _ in shape))


def _rows(width):
    return pl.BlockSpec((_RB, width), lambda i: (i, 0))


_pair_out = dict(
    out_specs=[_rows(128), _rows(128)],
    out_shape=[jax.ShapeDtypeStruct((N, 128), _f32),
               jax.ShapeDtypeStruct((N, 128), _f32)],
)

_t1 = pl.pallas_call(
    _t1_body,
    grid=(_NB,),
    in_specs=[_rows(128), _full((128, 256))],
    **_pair_out,
)

_rows_hi = pl.BlockSpec((_RB, 128), lambda i: (i + _NB, 0))

_t2 = pl.pallas_call(
    _t2_body,
    grid=(_NB,),
    in_specs=[_rows(128), _rows_hi, _full((1, 64)), _full((64, 256))],
    **_pair_out,
)

_t4 = pl.pallas_call(
    _t4_body,
    grid=(1,),
    in_specs=[pl.BlockSpec((N, 128), lambda i: (0, 0)),
              pl.BlockSpec((N, 128), lambda i: (1, 0)),
              _full((N, 1)),
              _full((1, 8)), _full((8, 16)), _full((1, 16))],
    out_specs=_full((G, OUT)),
    out_shape=jax.ShapeDtypeStruct((G, OUT), _f32),
)


def _compact_mat(a):
    hh = a.shape[0] * a.shape[1]
    hid = a.shape[1]
    ii = jnp.arange(hh)
    jj = jnp.arange(a.shape[0])
    mask = (ii[:, None] // hid) == jj[None, :]
    return mask.astype(_f32) * a.reshape(hh)[:, None]


def _expand_mat(a):
    hh = a.shape[0] * a.shape[1]
    hid = a.shape[1]
    ii = jnp.arange(hh)
    mask = (ii[:, None] // hid) == (ii[None, :] // hid)
    return mask.astype(_f32) * a.reshape(hh)[:, None]


def kernel(x, edge_index, batch, W1, a_s1, a_d1, b1, W2, a_s2, a_d2, b2,
           W3, a_s3, a_d3, b3, Wc, bc):
    src = edge_index[0]
    dst = edge_index[1]

    z64 = jnp.zeros((64, 64), _f32)
    w1c = jnp.concatenate(
        [W1, W1 @ _expand_mat(a_s1), W1 @ _expand_mat(a_d1),
         jnp.zeros((DIN, 64), _f32)], axis=1)
    w2c = jnp.concatenate(
        [W2, W2 @ _expand_mat(a_s2), W2 @ _expand_mat(a_d2), z64], axis=1)
    es3 = a_s3.reshape(HID, 1) @ jnp.ones((1, 64), _f32)
    ed3 = a_d3.reshape(HID, 1) @ jnp.ones((1, 64), _f32)
    w3c = jnp.concatenate(
        [jnp.tile(W3, (1, HEADS)), W3 @ es3, W3 @ ed3, z64], axis=1)

    sc_edge = _sc_kernel()

    hs1, ed1 = _t1(x, w1c)
    acc1 = sc_edge(src, dst, hs1, ed1)
    hs2, ed2 = _t2(acc1, acc1, b1.reshape(1, 64), w2c)
    acc2 = sc_edge(src, dst, hs2, ed2)
    hs3, ed3t = _t2(acc2, acc2, b2.reshape(1, 64), w3c)
    acc3 = sc_edge(src, dst, hs3, ed3t)
    out = _t4(acc3, acc3, batch.reshape(N, 1), b3.reshape(1, 8),
              Wc, bc.reshape(1, OUT))
    return out
